# Initial kernel scaffold; baseline (speedup 1.0000x reference)
import jax
import jax.numpy as jnp
from jax.experimental import pallas as pl


def kernel(x, edge_index, edge_weight, batch, W1, b1, W2, b2, W3, b3):
    raise NotImplementedError("write your pallas kernel here")


# speedup vs baseline: 3.3323x; 3.3323x over previous
import dataclasses
import functools

import jax
import jax.numpy as jnp
from jax import lax
from jax.experimental import pallas as pl
from jax.experimental.pallas import tpu as pltpu
from jax.experimental.pallas import tpu_sc as plsc

N = 10000
D = 128
NC = 2
NS = 16
LANES = 16
NW = NC * NS
CHUNK = 128
NPAD = 10240
RPT = NPAD // NS
NEG_SLOPE = 0.01
EPS = 1e-5

_SC_PARAMS = pltpu.CompilerParams()
if "needs_layout_passes" in pltpu.CompilerParams.__dataclass_fields__:
    _SC_PARAMS = dataclasses.replace(_SC_PARAMS, needs_layout_passes=False)


def _propagate_sc(h, src, dst, w, zeros, epad):
    ept = epad // NW
    nchunk = ept // CHUNK
    mesh = plsc.VectorSubcoreMesh(core_axis_name="c", subcore_axis_name="s")

    @functools.partial(
        pl.kernel,
        out_type=jax.ShapeDtypeStruct((NC, NPAD, D), jnp.float32),
        mesh=mesh,
        compiler_params=_SC_PARAMS,
        scratch_types=[
            pltpu.VMEM_SHARED((NPAD, D), jnp.float32),
            pltpu.VMEM((CHUNK,), jnp.int32),
            pltpu.VMEM((CHUNK,), jnp.int32),
            pltpu.VMEM((CHUNK,), jnp.float32),
            pltpu.VMEM((CHUNK, D), jnp.float32),
            pltpu.SemaphoreType.DMA,
            pltpu.SemaphoreType.DMA,
            pltpu.SemaphoreType.DMA,
            pltpu.SemaphoreType.DMA,
        ],
    )
    def k(h_hbm, src_hbm, dst_hbm, w_hbm, z_hbm, out_hbm,
          acc, src_v, dst_v, w_v, rows_v, sem0, sem1, sem2, sem3):
        c = lax.axis_index("c")
        s = lax.axis_index("s")
        wid = c * NS + s

        pltpu.sync_copy(z_hbm.at[pl.ds(s * RPT, RPT)],
                        acc.at[pl.ds(s * RPT, RPT)])
        plsc.subcore_barrier()

        base0 = wid * ept

        @pl.loop(0, nchunk)
        def _(ci):
            base = base0 + ci * CHUNK
            cp0 = pltpu.async_copy(src_hbm.at[pl.ds(base, CHUNK)], src_v, sem0)
            cp1 = pltpu.async_copy(dst_hbm.at[pl.ds(base, CHUNK)], dst_v, sem1)
            cp2 = pltpu.async_copy(w_hbm.at[pl.ds(base, CHUNK)], w_v, sem2)
            cp0.wait()
            cp1.wait()
            cp2.wait()
            pltpu.async_copy(h_hbm.at[src_v], rows_v, sem3).wait()

            @pl.loop(0, CHUNK)
            def _(e):
                idx = jnp.zeros((LANES,), jnp.int32) + e
                wb = plsc.load_gather(w_v, [idx])
                for j in range(D // LANES):
                    seg = rows_v[e, pl.ds(j * LANES, LANES)]
                    rows_v[e, pl.ds(j * LANES, LANES)] = seg * wb

            pltpu.sync_copy(rows_v, acc.at[dst_v], add=True)

        plsc.subcore_barrier()
        pltpu.sync_copy(acc.at[pl.ds(s * RPT, RPT)],
                        out_hbm.at[c, pl.ds(s * RPT, RPT)])

    return k(h, src, dst, w, zeros)


def _dense_tc(agg, h, W, b, bn):
    out_dim = W.shape[1]

    def body(agg_ref, h_ref, w_ref, b_ref, o_ref):
        a = agg_ref[0, :N, :] + agg_ref[1, :N, :] + h_ref[...]
        y = jnp.dot(a, w_ref[...], preferred_element_type=jnp.float32)
        y = y + b_ref[...]
        if bn:
            m = jnp.mean(y, axis=0, keepdims=True)
            v = jnp.mean((y - m) ** 2, axis=0, keepdims=True)
            y = (y - m) * lax.rsqrt(v + EPS)
            y = jnp.where(y >= 0.0, y, NEG_SLOPE * y)
        o_ref[...] = y

    return pl.pallas_call(
        body,
        out_shape=jax.ShapeDtypeStruct((N, out_dim), jnp.float32),
    )(agg, h, W, b.reshape(1, out_dim))


def kernel(x, edge_index, edge_weight, batch, W1, b1, W2, b2, W3, b3):
    e = edge_index.shape[1]
    epad = ((e + NW * CHUNK - 1) // (NW * CHUNK)) * (NW * CHUNK)
    pad = epad - e
    src = jnp.concatenate([edge_index[0], jnp.zeros((pad,), jnp.int32)])
    dst = jnp.concatenate([edge_index[1], jnp.zeros((pad,), jnp.int32)])
    w = jnp.concatenate([edge_weight, jnp.zeros((pad,), jnp.float32)])
    zeros = jnp.zeros((NPAD, D), jnp.float32)

    h = x
    agg = _propagate_sc(h, src, dst, w, zeros, epad)
    h = _dense_tc(agg, h, W1, b1, True)
    agg = _propagate_sc(h, src, dst, w, zeros, epad)
    h = _dense_tc(agg, h, W2, b2, True)
    agg = _propagate_sc(h, src, dst, w, zeros, epad)
    return _dense_tc(agg, h, W3, b3, False)
